# Initial kernel scaffold; baseline (speedup 1.0000x reference)
import jax
import jax.numpy as jnp
from jax.experimental import pallas as pl


def kernel(x, edge_index, W1, b1, W2, b2, W4, b4):
    raise NotImplementedError("write your pallas kernel here")


# speedup vs baseline: 17.8684x; 17.8684x over previous
import functools

import jax
import jax.numpy as jnp
from jax import lax
from jax.experimental import pallas as pl
from jax.experimental.pallas import tpu as pltpu
from jax.experimental.pallas import tpu_sc as plsc

N = 10000
E = 320000
N_PAD = 10240
E_PAD = 327680
NC = 2
NS = 16
NW = NC * NS
CHUNK = 128
CPW = E_PAD // (NW * CHUNK)
RPT = N_PAD // NS

_mesh = plsc.VectorSubcoreMesh(
    core_axis_name="c", subcore_axis_name="s", num_cores=NC, num_subcores=NS)
_sc_params = pltpu.CompilerParams(use_tc_tiling_on_sc=False)


@functools.partial(
    pl.kernel,
    out_type=jax.ShapeDtypeStruct((NC, N_PAD, 16), jnp.float32),
    mesh=_mesh,
    scratch_types=[
        pltpu.VMEM((CPW, CHUNK), jnp.int32),
        pltpu.VMEM((CHUNK, 16), jnp.float32),
        pltpu.VMEM_SHARED((N_PAD, 16), jnp.float32),
    ],
    compiler_params=_sc_params,
)
def _deg_kernel(dst_hbm, zeros16_hbm, ones16_hbm, out_hbm, idx_v, ones_v, acc_sh):
    c = lax.axis_index("c")
    s = lax.axis_index("s")
    wid = c * NS + s
    rbase = s * RPT
    pltpu.sync_copy(zeros16_hbm.at[pl.ds(rbase, RPT)], acc_sh.at[pl.ds(rbase, RPT)])
    pltpu.sync_copy(dst_hbm.at[pl.ds(wid * CPW, CPW)], idx_v)
    pltpu.sync_copy(ones16_hbm, ones_v)
    plsc.subcore_barrier()

    def body(j, carry):
        pltpu.sync_copy(ones_v, acc_sh.at[idx_v.at[j]], add=True)
        return carry

    lax.fori_loop(0, CPW, body, 0)
    plsc.subcore_barrier()
    pltpu.sync_copy(acc_sh.at[pl.ds(rbase, RPT)], out_hbm.at[c, pl.ds(rbase, RPT)])


@functools.partial(
    pl.kernel,
    out_type=jax.ShapeDtypeStruct((NC, N_PAD, 32), jnp.float32),
    mesh=_mesh,
    scratch_types=[
        pltpu.VMEM((CPW, CHUNK), jnp.int32),
        pltpu.VMEM((CPW, CHUNK), jnp.int32),
        pltpu.VMEM((CHUNK, 32), jnp.float32),
        pltpu.VMEM_SHARED((N_PAD, 32), jnp.float32),
        pltpu.SemaphoreType.DMA,
    ],
    compiler_params=_sc_params,
)
def _prop_kernel(table_hbm, src_hbm, dst_hbm, zeros32_hbm, out_hbm,
                 src_v, dst_v, rows_v, acc_sh, sem):
    c = lax.axis_index("c")
    s = lax.axis_index("s")
    wid = c * NS + s
    rbase = s * RPT
    pltpu.sync_copy(zeros32_hbm.at[pl.ds(rbase, RPT)], acc_sh.at[pl.ds(rbase, RPT)])
    pltpu.sync_copy(src_hbm.at[pl.ds(wid * CPW, CPW)], src_v)
    pltpu.sync_copy(dst_hbm.at[pl.ds(wid * CPW, CPW)], dst_v)
    plsc.subcore_barrier()

    def body(j, carry):
        pltpu.async_copy(table_hbm.at[src_v.at[j]], rows_v, sem).wait()
        pltpu.sync_copy(rows_v, acc_sh.at[dst_v.at[j]], add=True)
        return carry

    lax.fori_loop(0, CPW, body, 0)
    plsc.subcore_barrier()
    pltpu.sync_copy(acc_sh.at[pl.ds(rbase, RPT)], out_hbm.at[c, pl.ds(rbase, RPT)])


def _tc_stage1(degp_ref, x_ref, w1_ref, t1_ref, table_ref, dinv_ref):
    deg = degp_ref[0, :, 0:1] + degp_ref[1, :, 0:1] + 1.0
    dinv = lax.rsqrt(deg)
    t1 = jnp.dot(x_ref[...], w1_ref[...], preferred_element_type=jnp.float32)
    t1_ref[...] = t1
    table_ref[...] = t1 * dinv
    dinv_ref[...] = dinv


def _tc_stage2(pp_ref, t1_ref, dinv_ref, b1_ref, h1_ref, table_ref):
    dinv = dinv_ref[...]
    p = (pp_ref[0] + pp_ref[1]) * dinv + (dinv * dinv) * t1_ref[...]
    a = p + b1_ref[...]
    h1 = jnp.where(a > 0, a, jnp.exp(jnp.minimum(a, 0.0)) - 1.0)
    h1_ref[...] = h1
    table_ref[...] = h1 * dinv


def _tc_stage3(pp_ref, h1_ref, dinv_ref, w2_ref, b2_ref, w4_ref, t3_ref, table_ref):
    dinv = dinv_ref[...]
    p = (pp_ref[0] + pp_ref[1]) * dinv + (dinv * dinv) * h1_ref[...]
    a = jnp.dot(p, w2_ref[...], preferred_element_type=jnp.float32) + b2_ref[...]
    h2 = jnp.where(a > 0, a, jnp.exp(jnp.minimum(a, 0.0)) - 1.0)
    t3 = jnp.dot(h2, w4_ref[...], preferred_element_type=jnp.float32)
    t3_ref[...] = t3
    table_ref[...] = t3 * dinv


def _tc_stage4(pp_ref, t3_ref, dinv_ref, b4_ref, out_ref):
    dinv = dinv_ref[...]
    logits = (pp_ref[0] + pp_ref[1]) * dinv + (dinv * dinv) * t3_ref[...] + b4_ref[...]
    col = lax.broadcasted_iota(jnp.int32, logits.shape, 1)
    z = jnp.where(col < 19, logits, -jnp.inf)
    zmax = jnp.max(z, axis=1, keepdims=True)
    e = jnp.exp(z - zmax)
    out_ref[...] = e / jnp.sum(e, axis=1, keepdims=True)


def _sds(shape):
    return jax.ShapeDtypeStruct(shape, jnp.float32)


def kernel(x, edge_index, W1, b1, W2, b2, W4, b4):
    fill = jnp.full((E_PAD - E,), N_PAD - 1, jnp.int32)
    src2d = jnp.concatenate([edge_index[0], fill]).reshape(E_PAD // CHUNK, CHUNK)
    dst2d = jnp.concatenate([edge_index[1], fill]).reshape(E_PAD // CHUNK, CHUNK)
    xpad = jnp.pad(x, ((0, N_PAD - N), (0, 0)))
    zeros16 = jnp.zeros((N_PAD, 16), jnp.float32)
    zeros32 = jnp.zeros((N_PAD, 32), jnp.float32)
    ones16 = jnp.ones((CHUNK, 16), jnp.float32)
    W4p = jnp.zeros((64, 32), jnp.float32).at[:, :19].set(W4)
    b1r = b1.reshape(1, 32)
    b2r = b2.reshape(1, 64)
    b4r = jnp.zeros((1, 32), jnp.float32).at[0, :19].set(b4)

    degp = _deg_kernel(dst2d, zeros16, ones16)

    t1, table1, dinv = pl.pallas_call(
        _tc_stage1,
        out_shape=[_sds((N_PAD, 32)), _sds((N_PAD, 32)), _sds((N_PAD, 1))],
    )(degp, xpad, W1)

    pp1 = _prop_kernel(table1, src2d, dst2d, zeros32)

    h1, table2 = pl.pallas_call(
        _tc_stage2,
        out_shape=[_sds((N_PAD, 32)), _sds((N_PAD, 32))],
    )(pp1, t1, dinv, b1r)

    pp2 = _prop_kernel(table2, src2d, dst2d, zeros32)

    t3, table3 = pl.pallas_call(
        _tc_stage3,
        out_shape=[_sds((N_PAD, 32)), _sds((N_PAD, 32))],
    )(pp2, h1, dinv, W2, b2r, W4p)

    pp3 = _prop_kernel(table3, src2d, dst2d, zeros32)

    probs = pl.pallas_call(
        _tc_stage4,
        out_shape=_sds((N_PAD, 32)),
    )(pp3, t3, dinv, b4r)

    return probs[:N, :19]


# speedup vs baseline: 22.8223x; 1.2772x over previous
import functools

import jax
import jax.numpy as jnp
from jax import lax
from jax.experimental import pallas as pl
from jax.experimental.pallas import tpu as pltpu
from jax.experimental.pallas import tpu_sc as plsc

N = 10000
E = 320000
N_PAD = 10240
E_PAD = 327680
NC = 2
NS = 16
NW = NC * NS
CHUNK = 128
CPW = E_PAD // (NW * CHUNK)
RPT = N_PAD // NS
NBUF = 4

_mesh = plsc.VectorSubcoreMesh(
    core_axis_name="c", subcore_axis_name="s", num_cores=NC, num_subcores=NS)
_sc_params = pltpu.CompilerParams(use_tc_tiling_on_sc=False)


@functools.partial(
    pl.kernel,
    out_type=jax.ShapeDtypeStruct((NC, N_PAD, 16), jnp.float32),
    mesh=_mesh,
    scratch_types=[
        pltpu.VMEM((CPW, CHUNK), jnp.int32),
        pltpu.VMEM((CHUNK, 16), jnp.float32),
        pltpu.VMEM_SHARED((N_PAD, 16), jnp.float32),
        pltpu.SemaphoreType.DMA,
    ],
    compiler_params=_sc_params,
)
def _deg_kernel(dst_hbm, zeros16_hbm, ones16_hbm, out_hbm, idx_v, ones_v, acc_sh, sem):
    c = lax.axis_index("c")
    s = lax.axis_index("s")
    wid = c * NS + s
    rbase = s * RPT
    pltpu.sync_copy(zeros16_hbm.at[pl.ds(rbase, RPT)], acc_sh.at[pl.ds(rbase, RPT)])
    pltpu.sync_copy(dst_hbm.at[pl.ds(wid * CPW, CPW)], idx_v)
    pltpu.sync_copy(ones16_hbm, ones_v)
    plsc.subcore_barrier()

    def body(j, carry):
        pltpu.async_copy(ones_v, acc_sh.at[idx_v.at[j]], sem, add=True)
        return carry

    lax.fori_loop(0, CPW, body, 0)

    def drain(j, carry):
        pltpu.make_async_copy(ones_v, acc_sh.at[idx_v.at[j]], sem).wait()
        return carry

    lax.fori_loop(0, CPW, drain, 0)
    plsc.subcore_barrier()
    pltpu.sync_copy(acc_sh.at[pl.ds(rbase, RPT)], out_hbm.at[c, pl.ds(rbase, RPT)])


@functools.partial(
    pl.kernel,
    out_type=jax.ShapeDtypeStruct((NC, N_PAD, 32), jnp.float32),
    mesh=_mesh,
    scratch_types=[
        pltpu.VMEM((CPW, CHUNK), jnp.int32),
        pltpu.VMEM((CPW, CHUNK), jnp.int32),
        [pltpu.VMEM((CHUNK, 32), jnp.float32)] * NBUF,
        [pltpu.SemaphoreType.DMA] * NBUF,
        pltpu.VMEM_SHARED((N_PAD, 32), jnp.float32),
    ],
    compiler_params=_sc_params,
)
def _prop_kernel(table_hbm, src_hbm, dst_hbm, zeros32_hbm, out_hbm,
                 src_v, dst_v, rows, sems, acc_sh):
    c = lax.axis_index("c")
    s = lax.axis_index("s")
    wid = c * NS + s
    rbase = s * RPT
    pltpu.sync_copy(zeros32_hbm.at[pl.ds(rbase, RPT)], acc_sh.at[pl.ds(rbase, RPT)])
    pltpu.sync_copy(src_hbm.at[pl.ds(wid * CPW, CPW)], src_v)
    pltpu.sync_copy(dst_hbm.at[pl.ds(wid * CPW, CPW)], dst_v)
    plsc.subcore_barrier()

    for b in range(NBUF):
        pltpu.async_copy(table_hbm.at[src_v.at[b]], rows[b], sems[b])

    def outer(g, carry):
        base = g * NBUF
        for b in range(NBUF):
            j = base + b
            pltpu.make_async_copy(table_hbm.at[src_v.at[j]], rows[b], sems[b]).wait()
            pltpu.sync_copy(rows[b], acc_sh.at[dst_v.at[j]], add=True)
            pltpu.async_copy(table_hbm.at[src_v.at[j + NBUF]], rows[b], sems[b])
        return carry

    lax.fori_loop(0, CPW // NBUF - 1, outer, 0)

    for b in range(NBUF):
        j = CPW - NBUF + b
        pltpu.make_async_copy(table_hbm.at[src_v.at[j]], rows[b], sems[b]).wait()
        pltpu.sync_copy(rows[b], acc_sh.at[dst_v.at[j]], add=True)

    plsc.subcore_barrier()
    pltpu.sync_copy(acc_sh.at[pl.ds(rbase, RPT)], out_hbm.at[c, pl.ds(rbase, RPT)])


def _tc_stage1(degp_ref, x_ref, w1_ref, t1_ref, table_ref, dinv_ref):
    deg = degp_ref[0, :, 0:1] + degp_ref[1, :, 0:1] + 1.0
    dinv = lax.rsqrt(deg)
    t1 = jnp.dot(x_ref[...], w1_ref[...], preferred_element_type=jnp.float32)
    t1_ref[...] = t1
    table_ref[...] = t1 * dinv
    dinv_ref[...] = dinv


def _tc_stage2(pp_ref, t1_ref, dinv_ref, b1_ref, h1_ref, table_ref):
    dinv = dinv_ref[...]
    p = (pp_ref[0] + pp_ref[1]) * dinv + (dinv * dinv) * t1_ref[...]
    a = p + b1_ref[...]
    h1 = jnp.where(a > 0, a, jnp.exp(jnp.minimum(a, 0.0)) - 1.0)
    h1_ref[...] = h1
    table_ref[...] = h1 * dinv


def _tc_stage3(pp_ref, h1_ref, dinv_ref, w2_ref, b2_ref, w4_ref, t3_ref, table_ref):
    dinv = dinv_ref[...]
    p = (pp_ref[0] + pp_ref[1]) * dinv + (dinv * dinv) * h1_ref[...]
    a = jnp.dot(p, w2_ref[...], preferred_element_type=jnp.float32) + b2_ref[...]
    h2 = jnp.where(a > 0, a, jnp.exp(jnp.minimum(a, 0.0)) - 1.0)
    t3 = jnp.dot(h2, w4_ref[...], preferred_element_type=jnp.float32)
    t3_ref[...] = t3
    table_ref[...] = t3 * dinv


def _tc_stage4(pp_ref, t3_ref, dinv_ref, b4_ref, out_ref):
    dinv = dinv_ref[...]
    logits = (pp_ref[0] + pp_ref[1]) * dinv + (dinv * dinv) * t3_ref[...] + b4_ref[...]
    col = lax.broadcasted_iota(jnp.int32, logits.shape, 1)
    z = jnp.where(col < 19, logits, -jnp.inf)
    zmax = jnp.max(z, axis=1, keepdims=True)
    e = jnp.exp(z - zmax)
    out_ref[...] = e / jnp.sum(e, axis=1, keepdims=True)


def _sds(shape):
    return jax.ShapeDtypeStruct(shape, jnp.float32)


def kernel(x, edge_index, W1, b1, W2, b2, W4, b4):
    fill = jnp.full((E_PAD - E,), N_PAD - 1, jnp.int32)
    src2d = jnp.concatenate([edge_index[0], fill]).reshape(E_PAD // CHUNK, CHUNK)
    dst2d = jnp.concatenate([edge_index[1], fill]).reshape(E_PAD // CHUNK, CHUNK)
    xpad = jnp.pad(x, ((0, N_PAD - N), (0, 0)))
    zeros16 = jnp.zeros((N_PAD, 16), jnp.float32)
    zeros32 = jnp.zeros((N_PAD, 32), jnp.float32)
    ones16 = jnp.ones((CHUNK, 16), jnp.float32)
    W4p = jnp.zeros((64, 32), jnp.float32).at[:, :19].set(W4)
    b1r = b1.reshape(1, 32)
    b2r = b2.reshape(1, 64)
    b4r = jnp.zeros((1, 32), jnp.float32).at[0, :19].set(b4)

    degp = _deg_kernel(dst2d, zeros16, ones16)

    t1, table1, dinv = pl.pallas_call(
        _tc_stage1,
        out_shape=[_sds((N_PAD, 32)), _sds((N_PAD, 32)), _sds((N_PAD, 1))],
    )(degp, xpad, W1)

    pp1 = _prop_kernel(table1, src2d, dst2d, zeros32)

    h1, table2 = pl.pallas_call(
        _tc_stage2,
        out_shape=[_sds((N_PAD, 32)), _sds((N_PAD, 32))],
    )(pp1, t1, dinv, b1r)

    pp2 = _prop_kernel(table2, src2d, dst2d, zeros32)

    t3, table3 = pl.pallas_call(
        _tc_stage3,
        out_shape=[_sds((N_PAD, 32)), _sds((N_PAD, 32))],
    )(pp2, h1, dinv, W2, b2r, W4p)

    pp3 = _prop_kernel(table3, src2d, dst2d, zeros32)

    probs = pl.pallas_call(
        _tc_stage4,
        out_shape=_sds((N_PAD, 32)),
    )(pp3, t3, dinv, b4r)

    return probs[:N, :19]


# speedup vs baseline: 22.8597x; 1.0016x over previous
import functools

import jax
import jax.numpy as jnp
from jax import lax
from jax.experimental import pallas as pl
from jax.experimental.pallas import tpu as pltpu
from jax.experimental.pallas import tpu_sc as plsc

N = 10000
E = 320000
N_PAD = 10240
E_PAD = 327680
NC = 2
NS = 16
NW = NC * NS
CHUNK = 128
CPW = E_PAD // (NW * CHUNK)
RPT = N_PAD // NS
NB = 10
GD = 5

_mesh = plsc.VectorSubcoreMesh(
    core_axis_name="c", subcore_axis_name="s", num_cores=NC, num_subcores=NS)
_sc_params = pltpu.CompilerParams(use_tc_tiling_on_sc=False)


@functools.partial(
    pl.kernel,
    out_type=jax.ShapeDtypeStruct((NC, N_PAD, 16), jnp.float32),
    mesh=_mesh,
    scratch_types=[
        pltpu.VMEM((CPW, CHUNK), jnp.int32),
        pltpu.VMEM((CHUNK, 16), jnp.float32),
        pltpu.VMEM_SHARED((N_PAD, 16), jnp.float32),
        pltpu.SemaphoreType.DMA,
    ],
    compiler_params=_sc_params,
)
def _deg_kernel(dst_hbm, zeros16_hbm, ones16_hbm, out_hbm, idx_v, ones_v, acc_sh, sem):
    c = lax.axis_index("c")
    s = lax.axis_index("s")
    wid = c * NS + s
    rbase = s * RPT
    pltpu.sync_copy(zeros16_hbm.at[pl.ds(rbase, RPT)], acc_sh.at[pl.ds(rbase, RPT)])
    pltpu.sync_copy(dst_hbm.at[pl.ds(wid * CPW, CPW)], idx_v)
    pltpu.sync_copy(ones16_hbm, ones_v)
    plsc.subcore_barrier()

    def body(j, carry):
        pltpu.async_copy(ones_v, acc_sh.at[idx_v.at[j]], sem, add=True)
        return carry

    lax.fori_loop(0, CPW, body, 0)

    def drain(j, carry):
        pltpu.make_async_copy(ones_v, acc_sh.at[idx_v.at[j]], sem).wait()
        return carry

    lax.fori_loop(0, CPW, drain, 0)
    plsc.subcore_barrier()
    pltpu.sync_copy(acc_sh.at[pl.ds(rbase, RPT)], out_hbm.at[c, pl.ds(rbase, RPT)])


@functools.partial(
    pl.kernel,
    out_type=jax.ShapeDtypeStruct((NC, N_PAD, 32), jnp.float32),
    mesh=_mesh,
    scratch_types=[
        pltpu.VMEM((CPW, CHUNK), jnp.int32),
        pltpu.VMEM((CPW, CHUNK), jnp.int32),
        [pltpu.VMEM((CHUNK, 32), jnp.float32)] * NB,
        [pltpu.SemaphoreType.DMA] * NB,
        [pltpu.SemaphoreType.DMA] * NB,
        pltpu.VMEM_SHARED((N_PAD, 32), jnp.float32),
    ],
    compiler_params=_sc_params,
)
def _prop_kernel(table_hbm, src_hbm, dst_hbm, zeros32_hbm, out_hbm,
                 src_v, dst_v, rows, gsem, ssem, acc_sh):
    c = lax.axis_index("c")
    s = lax.axis_index("s")
    wid = c * NS + s
    rbase = s * RPT
    pltpu.sync_copy(zeros32_hbm.at[pl.ds(rbase, RPT)], acc_sh.at[pl.ds(rbase, RPT)])
    pltpu.sync_copy(src_hbm.at[pl.ds(wid * CPW, CPW)], src_v)
    pltpu.sync_copy(dst_hbm.at[pl.ds(wid * CPW, CPW)], dst_v)
    plsc.subcore_barrier()

    def gather(jt, b):
        pltpu.async_copy(table_hbm.at[src_v.at[jt]], rows[b], gsem[b])

    def gwait(j, b):
        pltpu.make_async_copy(table_hbm.at[src_v.at[j]], rows[b], gsem[b]).wait()

    def scatter(j, b):
        pltpu.async_copy(rows[b], acc_sh.at[dst_v.at[j]], ssem[b], add=True)

    def swait(j, b):
        pltpu.make_async_copy(rows[b], acc_sh.at[dst_v.at[j]], ssem[b]).wait()

    for b in range(GD):
        gather(b, b)

    for j in range(NB):
        b = j % NB
        gwait(j, b)
        scatter(j, b)
        jt = j + GD
        bt = jt % NB
        if jt >= NB:
            swait(jt - NB, bt)
        gather(jt, bt)

    def outer(g, carry):
        base = g * NB
        for b in range(NB):
            j = base + b
            gwait(j, b)
            scatter(j, b)
            jt = j + GD
            bt = (b + GD) % NB
            swait(jt - NB, bt)
            gather(jt, bt)
        return carry

    lax.fori_loop(1, CPW // NB - 1, outer, 0)

    for b in range(NB):
        j = CPW - NB + b
        gwait(j, b)
        scatter(j, b)
        jt = j + GD
        if jt < CPW:
            bt = jt % NB
            swait(jt - NB, bt)
            gather(jt, bt)

    for b in range(NB):
        swait(CPW - NB + b, b)

    plsc.subcore_barrier()
    pltpu.sync_copy(acc_sh.at[pl.ds(rbase, RPT)], out_hbm.at[c, pl.ds(rbase, RPT)])


def _tc_stage1(degp_ref, x_ref, w1_ref, t1_ref, table_ref, dinv_ref):
    deg = degp_ref[0, :, 0:1] + degp_ref[1, :, 0:1] + 1.0
    dinv = lax.rsqrt(deg)
    t1 = jnp.dot(x_ref[...], w1_ref[...], preferred_element_type=jnp.float32)
    t1_ref[...] = t1
    table_ref[...] = t1 * dinv
    dinv_ref[...] = dinv


def _tc_stage2(pp_ref, t1_ref, dinv_ref, b1_ref, h1_ref, table_ref):
    dinv = dinv_ref[...]
    p = (pp_ref[0] + pp_ref[1]) * dinv + (dinv * dinv) * t1_ref[...]
    a = p + b1_ref[...]
    h1 = jnp.where(a > 0, a, jnp.exp(jnp.minimum(a, 0.0)) - 1.0)
    h1_ref[...] = h1
    table_ref[...] = h1 * dinv


def _tc_stage3(pp_ref, h1_ref, dinv_ref, w2_ref, b2_ref, w4_ref, t3_ref, table_ref):
    dinv = dinv_ref[...]
    p = (pp_ref[0] + pp_ref[1]) * dinv + (dinv * dinv) * h1_ref[...]
    a = jnp.dot(p, w2_ref[...], preferred_element_type=jnp.float32) + b2_ref[...]
    h2 = jnp.where(a > 0, a, jnp.exp(jnp.minimum(a, 0.0)) - 1.0)
    t3 = jnp.dot(h2, w4_ref[...], preferred_element_type=jnp.float32)
    t3_ref[...] = t3
    table_ref[...] = t3 * dinv


def _tc_stage4(pp_ref, t3_ref, dinv_ref, b4_ref, out_ref):
    dinv = dinv_ref[...]
    logits = (pp_ref[0] + pp_ref[1]) * dinv + (dinv * dinv) * t3_ref[...] + b4_ref[...]
    col = lax.broadcasted_iota(jnp.int32, logits.shape, 1)
    z = jnp.where(col < 19, logits, -jnp.inf)
    zmax = jnp.max(z, axis=1, keepdims=True)
    e = jnp.exp(z - zmax)
    out_ref[...] = e / jnp.sum(e, axis=1, keepdims=True)


def _sds(shape):
    return jax.ShapeDtypeStruct(shape, jnp.float32)


def kernel(x, edge_index, W1, b1, W2, b2, W4, b4):
    fill = jnp.full((E_PAD - E,), N_PAD - 1, jnp.int32)
    src2d = jnp.concatenate([edge_index[0], fill]).reshape(E_PAD // CHUNK, CHUNK)
    dst2d = jnp.concatenate([edge_index[1], fill]).reshape(E_PAD // CHUNK, CHUNK)
    xpad = jnp.pad(x, ((0, N_PAD - N), (0, 0)))
    zeros16 = jnp.zeros((N_PAD, 16), jnp.float32)
    zeros32 = jnp.zeros((N_PAD, 32), jnp.float32)
    ones16 = jnp.ones((CHUNK, 16), jnp.float32)
    W4p = jnp.zeros((64, 32), jnp.float32).at[:, :19].set(W4)
    b1r = b1.reshape(1, 32)
    b2r = b2.reshape(1, 64)
    b4r = jnp.zeros((1, 32), jnp.float32).at[0, :19].set(b4)

    degp = _deg_kernel(dst2d, zeros16, ones16)

    t1, table1, dinv = pl.pallas_call(
        _tc_stage1,
        out_shape=[_sds((N_PAD, 32)), _sds((N_PAD, 32)), _sds((N_PAD, 1))],
    )(degp, xpad, W1)

    pp1 = _prop_kernel(table1, src2d, dst2d, zeros32)

    h1, table2 = pl.pallas_call(
        _tc_stage2,
        out_shape=[_sds((N_PAD, 32)), _sds((N_PAD, 32))],
    )(pp1, t1, dinv, b1r)

    pp2 = _prop_kernel(table2, src2d, dst2d, zeros32)

    t3, table3 = pl.pallas_call(
        _tc_stage3,
        out_shape=[_sds((N_PAD, 32)), _sds((N_PAD, 32))],
    )(pp2, h1, dinv, W2, b2r, W4p)

    pp3 = _prop_kernel(table3, src2d, dst2d, zeros32)

    probs = pl.pallas_call(
        _tc_stage4,
        out_shape=_sds((N_PAD, 32)),
    )(pp3, t3, dinv, b4r)

    return probs[:N, :19]


# speedup vs baseline: 40.0962x; 1.7540x over previous
import functools

import jax
import jax.numpy as jnp
from jax import lax
from jax.experimental import pallas as pl
from jax.experimental.pallas import tpu as pltpu
from jax.experimental.pallas import tpu_sc as plsc

N = 10000
E = 320000
N_PAD = 10240
E_PAD = 327680
NC = 2
NS = 16
NW = NC * NS
CHUNK = 128
CPW = E_PAD // (NW * CHUNK)
RPT = N_PAD // NS
NB = 10
GD = 5

_mesh = plsc.VectorSubcoreMesh(
    core_axis_name="c", subcore_axis_name="s", num_cores=NC, num_subcores=NS)
_sc_params = pltpu.CompilerParams(use_tc_tiling_on_sc=False)


@functools.partial(
    pl.kernel,
    out_type=jax.ShapeDtypeStruct((NC, N_PAD, 16), jnp.float32),
    mesh=_mesh,
    scratch_types=[
        pltpu.VMEM((CPW, CHUNK), jnp.int32),
        pltpu.VMEM((CHUNK, 16), jnp.float32),
        pltpu.VMEM_SHARED((N_PAD, 16), jnp.float32),
        pltpu.SemaphoreType.DMA,
    ],
    compiler_params=_sc_params,
)
def _deg_kernel(dst_hbm, zeros16_hbm, ones16_hbm, out_hbm, idx_v, ones_v, acc_sh, sem):
    c = lax.axis_index("c")
    s = lax.axis_index("s")
    wid = c * NS + s
    rbase = s * RPT
    pltpu.sync_copy(zeros16_hbm.at[pl.ds(rbase, RPT)], acc_sh.at[pl.ds(rbase, RPT)])
    pltpu.sync_copy(dst_hbm.at[pl.ds(wid * CPW, CPW)], idx_v)
    pltpu.sync_copy(ones16_hbm, ones_v)
    plsc.subcore_barrier()

    def body(j, carry):
        pltpu.async_copy(ones_v, acc_sh.at[idx_v.at[j]], sem, add=True)
        return carry

    lax.fori_loop(0, CPW, body, 0)

    def drain(j, carry):
        pltpu.make_async_copy(ones_v, acc_sh.at[idx_v.at[j]], sem).wait()
        return carry

    lax.fori_loop(0, CPW, drain, 0)
    plsc.subcore_barrier()
    pltpu.sync_copy(acc_sh.at[pl.ds(rbase, RPT)], out_hbm.at[c, pl.ds(rbase, RPT)])


@functools.partial(
    pl.kernel,
    out_type=jax.ShapeDtypeStruct((NC, N_PAD, 32), jnp.float32),
    mesh=_mesh,
    scratch_types=[
        pltpu.VMEM((CPW, CHUNK), jnp.int32),
        pltpu.VMEM((CPW, CHUNK), jnp.int32),
        [pltpu.VMEM((CHUNK, 32), jnp.float32)] * NB,
        [pltpu.SemaphoreType.DMA] * NB,
        [pltpu.SemaphoreType.DMA] * NB,
        pltpu.VMEM_SHARED((N_PAD, 32), jnp.float32),
        pltpu.VMEM_SHARED((N_PAD, 32), jnp.float32),
    ],
    compiler_params=_sc_params,
)
def _prop_kernel(table_hbm, src_hbm, dst_hbm, zeros32_hbm, out_hbm,
                 src_v, dst_v, rows, gsem, ssem, acc_sh, table_sh):
    c = lax.axis_index("c")
    s = lax.axis_index("s")
    wid = c * NS + s
    rbase = s * RPT
    pltpu.sync_copy(zeros32_hbm.at[pl.ds(rbase, RPT)], acc_sh.at[pl.ds(rbase, RPT)])
    pltpu.sync_copy(table_hbm.at[pl.ds(rbase, RPT)], table_sh.at[pl.ds(rbase, RPT)])
    pltpu.sync_copy(src_hbm.at[pl.ds(wid * CPW, CPW)], src_v)
    pltpu.sync_copy(dst_hbm.at[pl.ds(wid * CPW, CPW)], dst_v)
    plsc.subcore_barrier()

    def gather(jt, b):
        pltpu.async_copy(table_sh.at[src_v.at[jt]], rows[b], gsem[b])

    def gwait(j, b):
        pltpu.make_async_copy(table_sh.at[src_v.at[j]], rows[b], gsem[b]).wait()

    def scatter(j, b):
        pltpu.async_copy(rows[b], acc_sh.at[dst_v.at[j]], ssem[b], add=True)

    def swait(j, b):
        pltpu.make_async_copy(rows[b], acc_sh.at[dst_v.at[j]], ssem[b]).wait()

    for b in range(GD):
        gather(b, b)

    for j in range(NB):
        b = j % NB
        gwait(j, b)
        scatter(j, b)
        jt = j + GD
        bt = jt % NB
        if jt >= NB:
            swait(jt - NB, bt)
        gather(jt, bt)

    def outer(g, carry):
        base = g * NB
        for b in range(NB):
            j = base + b
            gwait(j, b)
            scatter(j, b)
            jt = j + GD
            bt = (b + GD) % NB
            swait(jt - NB, bt)
            gather(jt, bt)
        return carry

    lax.fori_loop(1, CPW // NB - 1, outer, 0)

    for b in range(NB):
        j = CPW - NB + b
        gwait(j, b)
        scatter(j, b)
        jt = j + GD
        if jt < CPW:
            bt = jt % NB
            swait(jt - NB, bt)
            gather(jt, bt)

    for b in range(NB):
        swait(CPW - NB + b, b)

    plsc.subcore_barrier()
    pltpu.sync_copy(acc_sh.at[pl.ds(rbase, RPT)], out_hbm.at[c, pl.ds(rbase, RPT)])


def _tc_stage1(degp_ref, x_ref, w1_ref, t1_ref, table_ref, dinv_ref):
    deg = degp_ref[0, :, 0:1] + degp_ref[1, :, 0:1] + 1.0
    dinv = lax.rsqrt(deg)
    t1 = jnp.dot(x_ref[...], w1_ref[...], preferred_element_type=jnp.float32)
    t1_ref[...] = t1
    table_ref[...] = t1 * dinv
    dinv_ref[...] = dinv


def _tc_stage2(pp_ref, t1_ref, dinv_ref, b1_ref, h1_ref, table_ref):
    dinv = dinv_ref[...]
    p = (pp_ref[0] + pp_ref[1]) * dinv + (dinv * dinv) * t1_ref[...]
    a = p + b1_ref[...]
    h1 = jnp.where(a > 0, a, jnp.exp(jnp.minimum(a, 0.0)) - 1.0)
    h1_ref[...] = h1
    table_ref[...] = h1 * dinv


def _tc_stage3(pp_ref, h1_ref, dinv_ref, w2_ref, b2_ref, w4_ref, t3_ref, table_ref):
    dinv = dinv_ref[...]
    p = (pp_ref[0] + pp_ref[1]) * dinv + (dinv * dinv) * h1_ref[...]
    a = jnp.dot(p, w2_ref[...], preferred_element_type=jnp.float32) + b2_ref[...]
    h2 = jnp.where(a > 0, a, jnp.exp(jnp.minimum(a, 0.0)) - 1.0)
    t3 = jnp.dot(h2, w4_ref[...], preferred_element_type=jnp.float32)
    t3_ref[...] = t3
    table_ref[...] = t3 * dinv


def _tc_stage4(pp_ref, t3_ref, dinv_ref, b4_ref, out_ref):
    dinv = dinv_ref[...]
    logits = (pp_ref[0] + pp_ref[1]) * dinv + (dinv * dinv) * t3_ref[...] + b4_ref[...]
    col = lax.broadcasted_iota(jnp.int32, logits.shape, 1)
    z = jnp.where(col < 19, logits, -jnp.inf)
    zmax = jnp.max(z, axis=1, keepdims=True)
    e = jnp.exp(z - zmax)
    out_ref[...] = e / jnp.sum(e, axis=1, keepdims=True)


def _sds(shape):
    return jax.ShapeDtypeStruct(shape, jnp.float32)


def kernel(x, edge_index, W1, b1, W2, b2, W4, b4):
    fill = jnp.full((E_PAD - E,), N_PAD - 1, jnp.int32)
    src2d = jnp.concatenate([edge_index[0], fill]).reshape(E_PAD // CHUNK, CHUNK)
    dst2d = jnp.concatenate([edge_index[1], fill]).reshape(E_PAD // CHUNK, CHUNK)
    xpad = jnp.pad(x, ((0, N_PAD - N), (0, 0)))
    zeros16 = jnp.zeros((N_PAD, 16), jnp.float32)
    zeros32 = jnp.zeros((N_PAD, 32), jnp.float32)
    ones16 = jnp.ones((CHUNK, 16), jnp.float32)
    W4p = jnp.zeros((64, 32), jnp.float32).at[:, :19].set(W4)
    b1r = b1.reshape(1, 32)
    b2r = b2.reshape(1, 64)
    b4r = jnp.zeros((1, 32), jnp.float32).at[0, :19].set(b4)

    degp = _deg_kernel(dst2d, zeros16, ones16)

    t1, table1, dinv = pl.pallas_call(
        _tc_stage1,
        out_shape=[_sds((N_PAD, 32)), _sds((N_PAD, 32)), _sds((N_PAD, 1))],
    )(degp, xpad, W1)

    pp1 = _prop_kernel(table1, src2d, dst2d, zeros32)

    h1, table2 = pl.pallas_call(
        _tc_stage2,
        out_shape=[_sds((N_PAD, 32)), _sds((N_PAD, 32))],
    )(pp1, t1, dinv, b1r)

    pp2 = _prop_kernel(table2, src2d, dst2d, zeros32)

    t3, table3 = pl.pallas_call(
        _tc_stage3,
        out_shape=[_sds((N_PAD, 32)), _sds((N_PAD, 32))],
    )(pp2, h1, dinv, W2, b2r, W4p)

    pp3 = _prop_kernel(table3, src2d, dst2d, zeros32)

    probs = pl.pallas_call(
        _tc_stage4,
        out_shape=_sds((N_PAD, 32)),
    )(pp3, t3, dinv, b4r)

    return probs[:N, :19]


# speedup vs baseline: 41.0244x; 1.0231x over previous
import functools

import jax
import jax.numpy as jnp
from jax import lax
from jax.experimental import pallas as pl
from jax.experimental.pallas import tpu as pltpu
from jax.experimental.pallas import tpu_sc as plsc

N = 10000
E = 320000
N_PAD = 10240
E_PAD = 327680
NC = 2
NS = 16
NW = NC * NS
CHUNK = 128
CPW = E_PAD // (NW * CHUNK)
RPT = N_PAD // NS
NB = 10
GD = 5

_mesh = plsc.VectorSubcoreMesh(
    core_axis_name="c", subcore_axis_name="s", num_cores=NC, num_subcores=NS)
_sc_params = pltpu.CompilerParams(use_tc_tiling_on_sc=False)


@functools.partial(
    pl.kernel,
    out_type=jax.ShapeDtypeStruct((NC, N_PAD, 16), jnp.float32),
    mesh=_mesh,
    scratch_types=[
        pltpu.VMEM((CPW, CHUNK), jnp.int32),
        pltpu.VMEM((CHUNK, 16), jnp.float32),
        pltpu.VMEM_SHARED((N_PAD, 16), jnp.float32),
        pltpu.SemaphoreType.DMA,
        pltpu.SemaphoreType.DMA,
        pltpu.SemaphoreType.DMA,
    ],
    compiler_params=_sc_params,
)
def _deg_kernel(dst_hbm, zeros16_hbm, ones16_hbm, out_hbm, idx_v, ones_v, acc_sh,
                sem, sem2, sem3):
    c = lax.axis_index("c")
    s = lax.axis_index("s")
    wid = c * NS + s
    rbase = s * RPT
    pltpu.async_copy(zeros16_hbm.at[pl.ds(rbase, RPT)], acc_sh.at[pl.ds(rbase, RPT)], sem)
    pltpu.async_copy(dst_hbm.at[pl.ds(wid * CPW, CPW)], idx_v, sem2)
    pltpu.async_copy(ones16_hbm, ones_v, sem3)
    pltpu.make_async_copy(zeros16_hbm.at[pl.ds(rbase, RPT)], acc_sh.at[pl.ds(rbase, RPT)], sem).wait()
    pltpu.make_async_copy(dst_hbm.at[pl.ds(wid * CPW, CPW)], idx_v, sem2).wait()
    pltpu.make_async_copy(ones16_hbm, ones_v, sem3).wait()
    plsc.subcore_barrier()

    def body(j, carry):
        pltpu.async_copy(ones_v, acc_sh.at[idx_v.at[j]], sem, add=True)
        return carry

    lax.fori_loop(0, CPW, body, 0)

    def drain(j, carry):
        pltpu.make_async_copy(ones_v, acc_sh.at[idx_v.at[j]], sem).wait()
        return carry

    lax.fori_loop(0, CPW, drain, 0)
    plsc.subcore_barrier()
    pltpu.sync_copy(acc_sh.at[pl.ds(rbase, RPT)], out_hbm.at[c, pl.ds(rbase, RPT)])


@functools.partial(
    pl.kernel,
    out_type=jax.ShapeDtypeStruct((NC, N_PAD, 32), jnp.float32),
    mesh=_mesh,
    scratch_types=[
        pltpu.VMEM((CPW, CHUNK), jnp.int32),
        pltpu.VMEM((CPW, CHUNK), jnp.int32),
        [pltpu.VMEM((CHUNK, 32), jnp.float32)] * NB,
        [pltpu.SemaphoreType.DMA] * NB,
        [pltpu.SemaphoreType.DMA] * NB,
        pltpu.VMEM_SHARED((N_PAD, 32), jnp.float32),
        pltpu.VMEM_SHARED((N_PAD, 32), jnp.float32),
    ],
    compiler_params=_sc_params,
)
def _prop_kernel(table_hbm, src_hbm, dst_hbm, zeros32_hbm, out_hbm,
                 src_v, dst_v, rows, gsem, ssem, acc_sh, table_sh):
    c = lax.axis_index("c")
    s = lax.axis_index("s")
    wid = c * NS + s
    rbase = s * RPT
    pltpu.async_copy(zeros32_hbm.at[pl.ds(rbase, RPT)], acc_sh.at[pl.ds(rbase, RPT)], gsem[0])
    pltpu.async_copy(table_hbm.at[pl.ds(rbase, RPT)], table_sh.at[pl.ds(rbase, RPT)], gsem[1])
    pltpu.async_copy(src_hbm.at[pl.ds(wid * CPW, CPW)], src_v, gsem[2])
    pltpu.async_copy(dst_hbm.at[pl.ds(wid * CPW, CPW)], dst_v, gsem[3])
    pltpu.make_async_copy(zeros32_hbm.at[pl.ds(rbase, RPT)], acc_sh.at[pl.ds(rbase, RPT)], gsem[0]).wait()
    pltpu.make_async_copy(table_hbm.at[pl.ds(rbase, RPT)], table_sh.at[pl.ds(rbase, RPT)], gsem[1]).wait()
    pltpu.make_async_copy(src_hbm.at[pl.ds(wid * CPW, CPW)], src_v, gsem[2]).wait()
    pltpu.make_async_copy(dst_hbm.at[pl.ds(wid * CPW, CPW)], dst_v, gsem[3]).wait()
    plsc.subcore_barrier()

    def gather(jt, b):
        pltpu.async_copy(table_sh.at[src_v.at[jt]], rows[b], gsem[b])

    def gwait(j, b):
        pltpu.make_async_copy(table_sh.at[src_v.at[j]], rows[b], gsem[b]).wait()

    def scatter(j, b):
        pltpu.async_copy(rows[b], acc_sh.at[dst_v.at[j]], ssem[b], add=True)

    def swait(j, b):
        pltpu.make_async_copy(rows[b], acc_sh.at[dst_v.at[j]], ssem[b]).wait()

    for b in range(GD):
        gather(b, b)

    for j in range(NB):
        b = j % NB
        gwait(j, b)
        scatter(j, b)
        jt = j + GD
        bt = jt % NB
        if jt >= NB:
            swait(jt - NB, bt)
        gather(jt, bt)

    def outer(g, carry):
        base = g * NB
        for b in range(NB):
            j = base + b
            gwait(j, b)
            scatter(j, b)
            jt = j + GD
            bt = (b + GD) % NB
            swait(jt - NB, bt)
            gather(jt, bt)
        return carry

    lax.fori_loop(1, CPW // NB - 1, outer, 0)

    for b in range(NB):
        j = CPW - NB + b
        gwait(j, b)
        scatter(j, b)
        jt = j + GD
        if jt < CPW:
            bt = jt % NB
            swait(jt - NB, bt)
            gather(jt, bt)

    for b in range(NB):
        swait(CPW - NB + b, b)

    plsc.subcore_barrier()
    pltpu.sync_copy(acc_sh.at[pl.ds(rbase, RPT)], out_hbm.at[c, pl.ds(rbase, RPT)])


def _tc_stage1(degp_ref, x_ref, w1_ref, t1_ref, table_ref, dinv_ref):
    deg = degp_ref[0, :, 0:1] + degp_ref[1, :, 0:1] + 1.0
    dinv = lax.rsqrt(deg)
    t1 = jnp.dot(x_ref[...], w1_ref[...], preferred_element_type=jnp.float32)
    t1_ref[...] = t1
    table_ref[...] = t1 * dinv
    dinv_ref[...] = dinv


def _tc_stage2(pp_ref, t1_ref, dinv_ref, b1_ref, h1_ref, table_ref):
    dinv = dinv_ref[...]
    p = (pp_ref[0] + pp_ref[1]) * dinv + (dinv * dinv) * t1_ref[...]
    a = p + b1_ref[...]
    h1 = jnp.where(a > 0, a, jnp.exp(jnp.minimum(a, 0.0)) - 1.0)
    h1_ref[...] = h1
    table_ref[...] = h1 * dinv


def _tc_stage3(pp_ref, h1_ref, dinv_ref, w2_ref, b2_ref, w4_ref, t3_ref, table_ref):
    dinv = dinv_ref[...]
    p = (pp_ref[0] + pp_ref[1]) * dinv + (dinv * dinv) * h1_ref[...]
    a = jnp.dot(p, w2_ref[...], preferred_element_type=jnp.float32) + b2_ref[...]
    h2 = jnp.where(a > 0, a, jnp.exp(jnp.minimum(a, 0.0)) - 1.0)
    t3 = jnp.dot(h2, w4_ref[...], preferred_element_type=jnp.float32)
    t3_ref[...] = t3
    table_ref[...] = t3 * dinv


def _tc_stage4(pp_ref, t3_ref, dinv_ref, b4_ref, out_ref):
    dinv = dinv_ref[...]
    logits = (pp_ref[0] + pp_ref[1]) * dinv + (dinv * dinv) * t3_ref[...] + b4_ref[...]
    col = lax.broadcasted_iota(jnp.int32, logits.shape, 1)
    z = jnp.where(col < 19, logits, -jnp.inf)
    zmax = jnp.max(z, axis=1, keepdims=True)
    e = jnp.exp(z - zmax)
    out_ref[...] = e / jnp.sum(e, axis=1, keepdims=True)


def _sds(shape):
    return jax.ShapeDtypeStruct(shape, jnp.float32)


def kernel(x, edge_index, W1, b1, W2, b2, W4, b4):
    fill = jnp.full((E_PAD - E,), N_PAD - 1, jnp.int32)
    src2d = jnp.concatenate([edge_index[0], fill]).reshape(E_PAD // CHUNK, CHUNK)
    dst2d = jnp.concatenate([edge_index[1], fill]).reshape(E_PAD // CHUNK, CHUNK)
    xpad = jnp.pad(x, ((0, N_PAD - N), (0, 0)))
    zeros16 = jnp.zeros((N_PAD, 16), jnp.float32)
    zeros32 = jnp.zeros((N_PAD, 32), jnp.float32)
    ones16 = jnp.ones((CHUNK, 16), jnp.float32)
    W4p = jnp.zeros((64, 32), jnp.float32).at[:, :19].set(W4)
    b1r = b1.reshape(1, 32)
    b2r = b2.reshape(1, 64)
    b4r = jnp.zeros((1, 32), jnp.float32).at[0, :19].set(b4)

    degp = _deg_kernel(dst2d, zeros16, ones16)

    t1, table1, dinv = pl.pallas_call(
        _tc_stage1,
        out_shape=[_sds((N_PAD, 32)), _sds((N_PAD, 32)), _sds((N_PAD, 1))],
    )(degp, xpad, W1)

    pp1 = _prop_kernel(table1, src2d, dst2d, zeros32)

    h1, table2 = pl.pallas_call(
        _tc_stage2,
        out_shape=[_sds((N_PAD, 32)), _sds((N_PAD, 32))],
    )(pp1, t1, dinv, b1r)

    pp2 = _prop_kernel(table2, src2d, dst2d, zeros32)

    t3, table3 = pl.pallas_call(
        _tc_stage3,
        out_shape=[_sds((N_PAD, 32)), _sds((N_PAD, 32))],
    )(pp2, h1, dinv, W2, b2r, W4p)

    pp3 = _prop_kernel(table3, src2d, dst2d, zeros32)

    probs = pl.pallas_call(
        _tc_stage4,
        out_shape=_sds((N_PAD, 32)),
    )(pp3, t3, dinv, b4r)

    return probs[:N, :19]


# speedup vs baseline: 42.3242x; 1.0317x over previous
import functools

import jax
import jax.numpy as jnp
from jax import lax
from jax.experimental import pallas as pl
from jax.experimental.pallas import tpu as pltpu
from jax.experimental.pallas import tpu_sc as plsc

N = 10000
E = 320000
N_PAD = 10240
E_PAD = 327680
NC = 2
NS = 16
NW = NC * NS
CHUNK = 128
CPW = E_PAD // (NW * CHUNK)
RPT = N_PAD // NS
NB = 10
GD = 5

_mesh = plsc.VectorSubcoreMesh(
    core_axis_name="c", subcore_axis_name="s", num_cores=NC, num_subcores=NS)
_sc_params = pltpu.CompilerParams(use_tc_tiling_on_sc=False)


@functools.partial(
    pl.kernel,
    out_type=jax.ShapeDtypeStruct((NC, N_PAD, 16), jnp.float32),
    mesh=_mesh,
    scratch_types=[
        pltpu.VMEM((CPW, CHUNK), jnp.int32),
        pltpu.VMEM((CHUNK, 16), jnp.float32),
        pltpu.VMEM_SHARED((N_PAD, 16), jnp.float32),
        pltpu.SemaphoreType.DMA,
        pltpu.SemaphoreType.DMA,
        pltpu.SemaphoreType.DMA,
    ],
    compiler_params=_sc_params,
)
def _deg_kernel(ei_hbm, zeros16_hbm, ones16_hbm, out_hbm, idx_v, ones_v, acc_sh,
                sem, sem2, sem3):
    c = lax.axis_index("c")
    s = lax.axis_index("s")
    wid = c * NS + s
    rbase = s * RPT
    pltpu.async_copy(zeros16_hbm.at[pl.ds(rbase, RPT)], acc_sh.at[pl.ds(rbase, RPT)], sem)
    pltpu.async_copy(ei_hbm.at[1, pl.ds(wid * CPW, CPW)], idx_v, sem2)
    pltpu.async_copy(ones16_hbm, ones_v, sem3)
    pltpu.make_async_copy(zeros16_hbm.at[pl.ds(rbase, RPT)], acc_sh.at[pl.ds(rbase, RPT)], sem).wait()
    pltpu.make_async_copy(ei_hbm.at[1, pl.ds(wid * CPW, CPW)], idx_v, sem2).wait()
    pltpu.make_async_copy(ones16_hbm, ones_v, sem3).wait()
    plsc.subcore_barrier()

    def body(j, carry):
        pltpu.async_copy(ones_v, acc_sh.at[idx_v.at[j]], sem, add=True)
        return carry

    lax.fori_loop(0, CPW, body, 0)

    def drain(j, carry):
        pltpu.make_async_copy(ones_v, acc_sh.at[idx_v.at[j]], sem).wait()
        return carry

    lax.fori_loop(0, CPW, drain, 0)
    plsc.subcore_barrier()
    pltpu.sync_copy(acc_sh.at[pl.ds(rbase, RPT)], out_hbm.at[c, pl.ds(rbase, RPT)])


@functools.partial(
    pl.kernel,
    out_type=jax.ShapeDtypeStruct((NC, N_PAD, 32), jnp.float32),
    mesh=_mesh,
    scratch_types=[
        pltpu.VMEM((CPW, CHUNK), jnp.int32),
        pltpu.VMEM((CPW, CHUNK), jnp.int32),
        [pltpu.VMEM((CHUNK, 32), jnp.float32)] * NB,
        [pltpu.SemaphoreType.DMA] * NB,
        [pltpu.SemaphoreType.DMA] * NB,
        pltpu.VMEM_SHARED((N_PAD, 32), jnp.float32),
        pltpu.VMEM_SHARED((N_PAD, 32), jnp.float32),
    ],
    compiler_params=_sc_params,
)
def _prop_kernel(table_hbm, ei_hbm, zeros32_hbm, out_hbm,
                 src_v, dst_v, rows, gsem, ssem, acc_sh, table_sh):
    c = lax.axis_index("c")
    s = lax.axis_index("s")
    wid = c * NS + s
    rbase = s * RPT
    pltpu.async_copy(zeros32_hbm.at[pl.ds(rbase, RPT)], acc_sh.at[pl.ds(rbase, RPT)], gsem[0])
    pltpu.async_copy(table_hbm.at[pl.ds(rbase, RPT)], table_sh.at[pl.ds(rbase, RPT)], gsem[1])
    pltpu.async_copy(ei_hbm.at[0, pl.ds(wid * CPW, CPW)], src_v, gsem[2])
    pltpu.async_copy(ei_hbm.at[1, pl.ds(wid * CPW, CPW)], dst_v, gsem[3])
    pltpu.make_async_copy(zeros32_hbm.at[pl.ds(rbase, RPT)], acc_sh.at[pl.ds(rbase, RPT)], gsem[0]).wait()
    pltpu.make_async_copy(table_hbm.at[pl.ds(rbase, RPT)], table_sh.at[pl.ds(rbase, RPT)], gsem[1]).wait()
    pltpu.make_async_copy(ei_hbm.at[0, pl.ds(wid * CPW, CPW)], src_v, gsem[2]).wait()
    pltpu.make_async_copy(ei_hbm.at[1, pl.ds(wid * CPW, CPW)], dst_v, gsem[3]).wait()
    plsc.subcore_barrier()

    def gather(jt, b):
        pltpu.async_copy(table_sh.at[src_v.at[jt]], rows[b], gsem[b])

    def gwait(j, b):
        pltpu.make_async_copy(table_sh.at[src_v.at[j]], rows[b], gsem[b]).wait()

    def scatter(j, b):
        pltpu.async_copy(rows[b], acc_sh.at[dst_v.at[j]], ssem[b], add=True)

    def swait(j, b):
        pltpu.make_async_copy(rows[b], acc_sh.at[dst_v.at[j]], ssem[b]).wait()

    for b in range(GD):
        gather(b, b)

    for j in range(NB):
        b = j % NB
        gwait(j, b)
        scatter(j, b)
        jt = j + GD
        bt = jt % NB
        if jt >= NB:
            swait(jt - NB, bt)
        gather(jt, bt)

    def outer(g, carry):
        base = g * NB
        for b in range(NB):
            j = base + b
            gwait(j, b)
            scatter(j, b)
            jt = j + GD
            bt = (b + GD) % NB
            swait(jt - NB, bt)
            gather(jt, bt)
        return carry

    lax.fori_loop(1, CPW // NB - 1, outer, 0)

    for b in range(NB):
        j = CPW - NB + b
        gwait(j, b)
        scatter(j, b)
        jt = j + GD
        if jt < CPW:
            bt = jt % NB
            swait(jt - NB, bt)
            gather(jt, bt)

    for b in range(NB):
        swait(CPW - NB + b, b)

    plsc.subcore_barrier()
    pltpu.sync_copy(acc_sh.at[pl.ds(rbase, RPT)], out_hbm.at[c, pl.ds(rbase, RPT)])


def _elu(a):
    return jnp.where(a > 0, a, jnp.exp(jnp.minimum(a, 0.0)) - 1.0)


def _tc_stage1(degp_hbm, x_ref, w1_ref, t1_ref, table_hbm, dinv_ref,
               degp_v, tbl_v, sem, sem2):
    pltpu.async_copy(degp_hbm, degp_v, sem).wait()
    deg = degp_v[0, 0:N, 0:1] + degp_v[1, 0:N, 0:1] + 1.0
    dinv = lax.rsqrt(deg)
    dinv_ref[...] = dinv
    t1 = jnp.dot(x_ref[...], w1_ref[...], preferred_element_type=jnp.float32)
    t1_ref[...] = t1
    tbl_v[...] = t1 * dinv
    pltpu.async_copy(tbl_v, table_hbm.at[pl.ds(0, N)], sem2).wait()


def _tc_stage2(pp_hbm, t1_ref, dinv_ref, b1_ref, h1_ref, table_hbm,
               pp_v, tbl_v, sem, sem2):
    pltpu.async_copy(pp_hbm, pp_v, sem).wait()
    dinv = dinv_ref[...]
    p = (pp_v[0, 0:N] + pp_v[1, 0:N]) * dinv + (dinv * dinv) * t1_ref[...]
    h1 = _elu(p + b1_ref[...])
    h1_ref[...] = h1
    tbl_v[...] = h1 * dinv
    pltpu.async_copy(tbl_v, table_hbm.at[pl.ds(0, N)], sem2).wait()


def _tc_stage3(pp_hbm, h1_ref, dinv_ref, w2_ref, b2_ref, w4_ref, t3_ref, table_hbm,
               pp_v, tbl_v, sem, sem2):
    pltpu.async_copy(pp_hbm, pp_v, sem).wait()
    dinv = dinv_ref[...]
    p = (pp_v[0, 0:N] + pp_v[1, 0:N]) * dinv + (dinv * dinv) * h1_ref[...]
    h2 = _elu(jnp.dot(p, w2_ref[...], preferred_element_type=jnp.float32) + b2_ref[...])
    t3 = jnp.dot(h2, w4_ref[...], preferred_element_type=jnp.float32)
    t3_ref[...] = t3
    tbl_v[...] = t3 * dinv
    pltpu.async_copy(tbl_v, table_hbm.at[pl.ds(0, N)], sem2).wait()


def _tc_stage4(pp_hbm, t3_ref, dinv_ref, b4_ref, out_ref, pp_v, sem):
    pltpu.async_copy(pp_hbm, pp_v, sem).wait()
    dinv = dinv_ref[...]
    logits = (pp_v[0, 0:N] + pp_v[1, 0:N]) * dinv + (dinv * dinv) * t3_ref[...] + b4_ref[...]
    col = lax.broadcasted_iota(jnp.int32, logits.shape, 1)
    z = jnp.where(col < 19, logits, -jnp.inf)
    zmax = jnp.max(z, axis=1, keepdims=True)
    e = jnp.exp(z - zmax)
    out_ref[...] = (e / jnp.sum(e, axis=1, keepdims=True))[:, 0:19]


def _sds(shape):
    return jax.ShapeDtypeStruct(shape, jnp.float32)


_HBM_SPEC = pl.BlockSpec(memory_space=pltpu.HBM)
_VMEM_SPEC = pl.BlockSpec(memory_space=pltpu.VMEM)


def kernel(x, edge_index, W1, b1, W2, b2, W4, b4):
    ei3d = jnp.pad(edge_index, ((0, 0), (0, E_PAD - E)),
                   constant_values=N_PAD - 1).reshape(2, E_PAD // CHUNK, CHUNK)
    zeros16 = jnp.zeros((N_PAD, 16), jnp.float32)
    zeros32 = jnp.zeros((N_PAD, 32), jnp.float32)
    ones16 = jnp.ones((CHUNK, 16), jnp.float32)
    W4p = jnp.zeros((64, 32), jnp.float32).at[:, :19].set(W4)
    b1r = b1.reshape(1, 32)
    b2r = b2.reshape(1, 64)
    b4r = jnp.zeros((1, 32), jnp.float32).at[0, :19].set(b4)

    degp = _deg_kernel(ei3d, zeros16, ones16)

    t1, table1, dinv = pl.pallas_call(
        _tc_stage1,
        out_shape=[_sds((N, 32)), _sds((N_PAD, 32)), _sds((N, 1))],
        in_specs=[_HBM_SPEC, _VMEM_SPEC, _VMEM_SPEC],
        out_specs=[_VMEM_SPEC, _HBM_SPEC, _VMEM_SPEC],
        scratch_shapes=[pltpu.VMEM((NC, N_PAD, 16), jnp.float32),
                        pltpu.VMEM((N, 32), jnp.float32),
                        pltpu.SemaphoreType.DMA, pltpu.SemaphoreType.DMA],
    )(degp, x, W1)

    pp1 = _prop_kernel(table1, ei3d, zeros32)

    h1, table2 = pl.pallas_call(
        _tc_stage2,
        out_shape=[_sds((N, 32)), _sds((N_PAD, 32))],
        in_specs=[_HBM_SPEC, _VMEM_SPEC, _VMEM_SPEC, _VMEM_SPEC],
        out_specs=[_VMEM_SPEC, _HBM_SPEC],
        scratch_shapes=[pltpu.VMEM((NC, N_PAD, 32), jnp.float32),
                        pltpu.VMEM((N, 32), jnp.float32),
                        pltpu.SemaphoreType.DMA, pltpu.SemaphoreType.DMA],
    )(pp1, t1, dinv, b1r)

    pp2 = _prop_kernel(table2, ei3d, zeros32)

    t3, table3 = pl.pallas_call(
        _tc_stage3,
        out_shape=[_sds((N, 32)), _sds((N_PAD, 32))],
        in_specs=[_HBM_SPEC] + [_VMEM_SPEC] * 5,
        out_specs=[_VMEM_SPEC, _HBM_SPEC],
        scratch_shapes=[pltpu.VMEM((NC, N_PAD, 32), jnp.float32),
                        pltpu.VMEM((N, 32), jnp.float32),
                        pltpu.SemaphoreType.DMA, pltpu.SemaphoreType.DMA],
    )(pp2, h1, dinv, W2, b2r, W4p)

    pp3 = _prop_kernel(table3, ei3d, zeros32)

    probs = pl.pallas_call(
        _tc_stage4,
        out_shape=_sds((N, 19)),
        in_specs=[_HBM_SPEC, _VMEM_SPEC, _VMEM_SPEC, _VMEM_SPEC],
        out_specs=_VMEM_SPEC,
        scratch_shapes=[pltpu.VMEM((NC, N_PAD, 32), jnp.float32),
                        pltpu.SemaphoreType.DMA],
    )(pp3, t3, dinv, b4r)

    return probs
